# Initial kernel scaffold; baseline (speedup 1.0000x reference)
import jax
import jax.numpy as jnp
from jax.experimental import pallas as pl


def kernel(xyz, eula_angle, edge_nearby, meta_type, fea, params):
    raise NotImplementedError("write your pallas kernel here")


# speedup vs baseline: 6.0851x; 6.0851x over previous
import functools

import jax
import jax.numpy as jnp
from jax import lax
from jax.experimental import pallas as pl
from jax.experimental.pallas import tpu as pltpu
from jax.experimental.pallas import tpu_sc as plsc

_NCTR = 512
_KNN = 32
_TBL_D = 80


def _fps_body(x_ref, far0_ref, out_ref):
    _, b, n = x_ref.shape
    iota = lax.broadcasted_iota(jnp.int32, (b, n), 1)
    col = lax.broadcasted_iota(jnp.int32, (b, _NCTR), 1)

    def step(i, carry):
        far, dist, acc = carry
        acc = jnp.where(col == i, far, acc)
        sel = (iota == far).astype(jnp.float32)
        x0 = x_ref[0]
        x1 = x_ref[1]
        x2 = x_ref[2]
        c0 = jnp.sum(x0 * sel, axis=1, keepdims=True)
        c1 = jnp.sum(x1 * sel, axis=1, keepdims=True)
        c2 = jnp.sum(x2 * sel, axis=1, keepdims=True)
        d = (x0 - c0) ** 2 + (x1 - c1) ** 2 + (x2 - c2) ** 2
        dist = jnp.minimum(dist, d)
        m = jnp.max(dist, axis=1, keepdims=True)
        far = jnp.min(jnp.where(dist == m, iota, n), axis=1,
                      keepdims=True)
        return far, dist, acc

    far0 = far0_ref[...]
    init = (far0, jnp.full((b, n), 1e10, jnp.float32),
            jnp.zeros((b, _NCTR), jnp.int32))
    _, _, acc = lax.fori_loop(0, _NCTR, step, init)
    out_ref[...] = acc


def _fps(xyz, far0):
    b, n, _ = xyz.shape
    xt = jnp.transpose(xyz, (2, 0, 1))
    return pl.pallas_call(
        _fps_body,
        out_shape=jax.ShapeDtypeStruct((b, _NCTR), jnp.int32),
    )(xt, far0)


def _knn_body(xyz_ref, xyzt_ref, fps_ref, out_ref, d_ref):
    n = xyz_ref.shape[1]
    xyz = xyz_ref[0]
    xt = xyzt_ref[0]
    fpsc = fps_ref[0]
    oh = (lax.broadcasted_iota(jnp.int32, (_NCTR, n), 1) == fpsc
          ).astype(jnp.float32)
    sq_col = jnp.sum(xyz * xyz, axis=1, keepdims=True)
    sq_row = jnp.sum(xt * xt, axis=0, keepdims=True)
    cxyz = jnp.dot(oh, xyz, preferred_element_type=jnp.float32)
    csq = jnp.dot(oh, sq_col, preferred_element_type=jnp.float32)
    dm = jnp.dot(cxyz, xt, preferred_element_type=jnp.float32)
    d_ref[...] = csq + sq_row - 2.0 * dm
    iota = lax.broadcasted_iota(jnp.int32, (_NCTR, n), 1)
    kio = lax.broadcasted_iota(jnp.int32, (_NCTR, _KNN), 1)

    def step(k, acc):
        d = d_ref[...]
        m = jnp.min(d, axis=1, keepdims=True)
        ik = jnp.min(jnp.where(d == m, iota, n), axis=1, keepdims=True)
        acc = jnp.where(kio == k, ik, acc)
        d_ref[...] = jnp.where(iota == ik, jnp.inf, d)
        return acc

    acc = lax.fori_loop(0, _KNN, step, jnp.zeros((_NCTR, _KNN), jnp.int32))
    out_ref[0] = acc


def _knn(xyz, fps):
    b, n, _ = xyz.shape
    xyzt = jnp.transpose(xyz, (0, 2, 1))
    return pl.pallas_call(
        _knn_body,
        grid=(b,),
        in_specs=[
            pl.BlockSpec((1, n, 3), lambda i: (i, 0, 0)),
            pl.BlockSpec((1, 3, n), lambda i: (i, 0, 0)),
            pl.BlockSpec((1, _NCTR, 1), lambda i: (i, 0, 0)),
        ],
        out_specs=pl.BlockSpec((1, _NCTR, _KNN), lambda i: (i, 0, 0)),
        out_shape=jax.ShapeDtypeStruct((b, _NCTR, _KNN), jnp.int32),
        scratch_shapes=[pltpu.VMEM((_NCTR, n), jnp.float32)],
    )(xyz, xyzt, fps.reshape(b, _NCTR, 1))


def _gather_rows(table, idx):
    q = idx.shape[0]
    d = table.shape[1]
    info = plsc.get_sparse_core_info()
    nw = info.num_cores * info.num_subcores
    ch = 128
    per_w = q // nw
    n_ch = per_w // ch
    mesh = plsc.VectorSubcoreMesh(core_axis_name="c", subcore_axis_name="s")

    @functools.partial(
        pl.kernel, mesh=mesh,
        out_type=jax.ShapeDtypeStruct((q, d), jnp.float32),
        compiler_params=pltpu.CompilerParams(use_tc_tiling_on_sc=False),
        scratch_types=[
            pltpu.VMEM((ch,), jnp.int32),
            pltpu.VMEM((ch, d), jnp.float32),
            pltpu.SemaphoreType.DMA,
        ],
    )
    def k(table_hbm, idx_hbm, out_hbm, idx_v, rows_v, sem):
        wid = lax.axis_index("s") * info.num_cores + lax.axis_index("c")
        base = wid * per_w

        def body(j, carry):
            off = base + j * ch
            pltpu.sync_copy(idx_hbm.at[pl.ds(off, ch)], idx_v)
            pltpu.async_copy(table_hbm.at[idx_v], rows_v, sem).wait()
            pltpu.sync_copy(rows_v, out_hbm.at[pl.ds(off, ch)])
            return carry

        lax.fori_loop(0, n_ch, body, 0)

    return k(table, idx)


def _acc_stats(y, s_ref, q_ref):
    @pl.when(pl.program_id(0) == 0)
    def _():
        s_ref[...] = jnp.zeros_like(s_ref)
        q_ref[...] = jnp.zeros_like(q_ref)

    s_ref[0:1, :] += jnp.sum(y, axis=0, keepdims=True)
    q_ref[0:1, :] += jnp.sum(y * y, axis=0, keepdims=True)


def _stats_outs(m, n_out):
    shapes = [jax.ShapeDtypeStruct((8, n_out), jnp.float32)] * 2
    specs = [pl.BlockSpec((8, n_out), lambda i: (0, 0))] * 2
    return shapes, specs


def _scale_shift(s, q, g, be, m):
    mean = s[0] / m
    var = q[0] / m - mean * mean
    sc = g / jnp.sqrt(var + 1e-5)
    sh = be - mean * sc
    return sc.reshape(1, -1), sh.reshape(1, -1)


def _mm(x, w, b, scale=None, shift=None, stats=False, bm=512):
    m, k = x.shape
    n = w.shape[1]
    affine = scale is not None

    def body(*refs):
        if affine:
            x_ref, sc_ref, sh_ref, w_ref, b_ref, y_ref, *st = refs
        else:
            x_ref, w_ref, b_ref, y_ref, *st = refs
        xv = x_ref[...]
        if affine:
            xv = jnp.maximum(xv * sc_ref[...] + sh_ref[...], 0.0)
        y = jnp.dot(xv, w_ref[...],
                    preferred_element_type=jnp.float32) + b_ref[...]
        y_ref[...] = y
        if stats:
            _acc_stats(y, *st)

    outs = [jax.ShapeDtypeStruct((m, n), jnp.float32)]
    out_specs = [pl.BlockSpec((bm, n), lambda i: (i, 0))]
    if stats:
        so, sp = _stats_outs(m, n)
        outs += so
        out_specs += sp
    ins = [x] + ([scale, shift] if affine else []) + [w, b.reshape(1, n)]
    in_specs = [pl.BlockSpec((bm, k), lambda i: (i, 0))]
    if affine:
        in_specs += [pl.BlockSpec((1, k), lambda i: (0, 0))] * 2
    in_specs += [pl.BlockSpec((k, n), lambda i: (0, 0)),
                 pl.BlockSpec((1, n), lambda i: (0, 0))]
    res = pl.pallas_call(body, grid=(m // bm,), in_specs=in_specs,
                         out_specs=out_specs, out_shape=outs)(*ins)
    return res if stats else res[0]


def _assemble_group(gk, gc):
    return jnp.concatenate([
        gk[:, 0:6] - gc[:, 0:6],
        gk[:, 6:8], gc[:, 6:8],
        gk[:, 8:12], gc[:, 8:12],
        gk[:, 12:76],
    ], axis=1)


def _assemble_cfa(gc):
    z = jnp.zeros_like(gc[:, 0:6])
    return jnp.concatenate([
        z, gc[:, 6:8], gc[:, 6:8], gc[:, 8:12], gc[:, 8:12], gc[:, 12:76],
    ], axis=1)


def _mm_first(gk, gc, w, b, cfa, bm=512):
    m = gk.shape[0] if not cfa else gc.shape[0]
    n = w.shape[1]
    k = w.shape[0]

    def body(*refs):
        if cfa:
            gc_ref, w_ref, b_ref, y_ref, s_ref, q_ref = refs
            xv = _assemble_cfa(gc_ref[...])
        else:
            gk_ref, gc_ref, w_ref, b_ref, y_ref, s_ref, q_ref = refs
            xv = _assemble_group(gk_ref[...], gc_ref[...])
        y = jnp.dot(xv, w_ref[...],
                    preferred_element_type=jnp.float32) + b_ref[...]
        y_ref[...] = y
        _acc_stats(y, s_ref, q_ref)

    so, sp = _stats_outs(m, n)
    outs = [jax.ShapeDtypeStruct((m, n), jnp.float32)] + so
    out_specs = [pl.BlockSpec((bm, n), lambda i: (i, 0))] + sp
    ins = ([gc] if cfa else [gk, gc]) + [w, b.reshape(1, n)]
    in_specs = [pl.BlockSpec((bm, _TBL_D), lambda i: (i, 0))] * (1 if cfa else 2)
    in_specs += [pl.BlockSpec((k, n), lambda i: (0, 0)),
                 pl.BlockSpec((1, n), lambda i: (0, 0))]
    return pl.pallas_call(body, grid=(m // bm,), in_specs=in_specs,
                          out_specs=out_specs, out_shape=outs)(*ins)


def _mm_gamma(fai, psi, w, b, bm=512):
    m, kdim = psi.shape
    n = w.shape[1]
    gpb = bm // _KNN

    def body(fai_ref, psi_ref, w_ref, b_ref, y_ref, s_ref, q_ref):
        e = (lax.broadcasted_iota(jnp.int32, (bm, gpb), 0) // _KNN
             == lax.broadcasted_iota(jnp.int32, (bm, gpb), 1)
             ).astype(jnp.float32)
        fai_e = jnp.dot(e, fai_ref[...], preferred_element_type=jnp.float32)
        xv = fai_e - psi_ref[...]
        y = jnp.dot(xv, w_ref[...],
                    preferred_element_type=jnp.float32) + b_ref[...]
        y_ref[...] = y
        _acc_stats(y, s_ref, q_ref)

    so, sp = _stats_outs(m, n)
    outs = [jax.ShapeDtypeStruct((m, n), jnp.float32)] + so
    out_specs = [pl.BlockSpec((bm, n), lambda i: (i, 0))] + sp
    in_specs = [
        pl.BlockSpec((gpb, kdim), lambda i: (i, 0)),
        pl.BlockSpec((bm, kdim), lambda i: (i, 0)),
        pl.BlockSpec((kdim, n), lambda i: (0, 0)),
        pl.BlockSpec((1, n), lambda i: (0, 0)),
    ]
    return pl.pallas_call(body, grid=(m // bm,), in_specs=in_specs,
                          out_specs=out_specs, out_shape=outs)(
        fai, psi, w, b.reshape(1, n))


def _mm_attn(g1, scale, shift, alpha, w, b, bm=512):
    m, kdim = g1.shape
    n = w.shape[1]
    gpb = bm // _KNN

    def body(g_ref, sc_ref, sh_ref, al_ref, w_ref, b_ref, y_ref):
        xv = jnp.maximum(g_ref[...] * sc_ref[...] + sh_ref[...], 0.0)
        gam = jnp.dot(xv, w_ref[...],
                      preferred_element_type=jnp.float32) + b_ref[...]
        mx = jnp.max(gam, axis=1, keepdims=True)
        ex = jnp.exp(gam - mx)
        sm = ex / jnp.sum(ex, axis=1, keepdims=True)
        t = float(n) * sm * al_ref[...]
        r = (lax.broadcasted_iota(jnp.int32, (gpb, bm), 1) // _KNN
             == lax.broadcasted_iota(jnp.int32, (gpb, bm), 0)
             ).astype(jnp.float32)
        y_ref[...] = jnp.dot(r, t, preferred_element_type=jnp.float32) / _KNN

    return pl.pallas_call(
        body, grid=(m // bm,),
        in_specs=[
            pl.BlockSpec((bm, kdim), lambda i: (i, 0)),
            pl.BlockSpec((1, kdim), lambda i: (0, 0)),
            pl.BlockSpec((1, kdim), lambda i: (0, 0)),
            pl.BlockSpec((bm, n), lambda i: (i, 0)),
            pl.BlockSpec((kdim, n), lambda i: (0, 0)),
            pl.BlockSpec((1, n), lambda i: (0, 0)),
        ],
        out_specs=pl.BlockSpec((gpb, n), lambda i: (i, 0)),
        out_shape=jax.ShapeDtypeStruct((m // _KNN, n), jnp.float32),
    )(g1, scale, shift, alpha, w, b.reshape(1, n))


def _mm_x3_psi1(y3, scale, shift, w, b, bm=512):
    m, kdim = y3.shape
    n = w.shape[1]

    def body(y3_ref, sc_ref, sh_ref, w_ref, b_ref, x3_ref, p1_ref,
             s_ref, q_ref):
        xv = jnp.maximum(y3_ref[...] * sc_ref[...] + sh_ref[...], 0.0)
        x3_ref[...] = xv
        y = jnp.dot(xv, w_ref[...],
                    preferred_element_type=jnp.float32) + b_ref[...]
        p1_ref[...] = y
        _acc_stats(y, s_ref, q_ref)

    so, sp = _stats_outs(m, n)
    outs = [jax.ShapeDtypeStruct((m, kdim), jnp.float32),
            jax.ShapeDtypeStruct((m, n), jnp.float32)] + so
    out_specs = [pl.BlockSpec((bm, kdim), lambda i: (i, 0)),
                 pl.BlockSpec((bm, n), lambda i: (i, 0))] + sp
    return pl.pallas_call(
        body, grid=(m // bm,),
        in_specs=[
            pl.BlockSpec((bm, kdim), lambda i: (i, 0)),
            pl.BlockSpec((1, kdim), lambda i: (0, 0)),
            pl.BlockSpec((1, kdim), lambda i: (0, 0)),
            pl.BlockSpec((kdim, n), lambda i: (0, 0)),
            pl.BlockSpec((1, n), lambda i: (0, 0)),
        ],
        out_specs=out_specs, out_shape=outs,
    )(y3, scale, shift, w, b.reshape(1, n))


def kernel(xyz, eula_angle, edge_nearby, meta_type, fea, params):
    b, n, _ = xyz.shape
    f32 = jnp.float32

    far0 = jax.random.randint(jax.random.key(42), (b,), 0, n
                              ).astype(jnp.int32).reshape(b, 1)
    fps = _fps(xyz, far0)
    idx = _knn(xyz, fps)

    table = jnp.concatenate(
        [xyz, eula_angle, edge_nearby, meta_type, fea,
         jnp.zeros((b, n, _TBL_D - 76), f32)], axis=-1).reshape(b * n, _TBL_D)
    off = (jnp.arange(b, dtype=jnp.int32) * n).reshape(b, 1)
    fps_g = fps + off
    idx_c = fps_g.reshape(-1)
    idx_k = (idx + off.reshape(b, 1, 1)).reshape(-1)
    idx_ce = jnp.broadcast_to(fps_g[:, :, None],
                              (b, _NCTR, _KNN)).reshape(-1)
    all_idx = jnp.concatenate([idx_c, idx_k, idx_ce]).astype(jnp.int32)

    g = _gather_rows(table, all_idx)
    mc = b * _NCTR
    mg = mc * _KNN
    gc0 = g[:mc]
    gk = g[mc:mc + mg]
    gce = g[mc + mg:]

    mlp = params['mlp']
    att = params['att']

    y1, s, q = _mm_first(gk, gce, mlp[0]['w'].T, mlp[0]['b'], cfa=False)
    sc1, sh1 = _scale_shift(s, q, mlp[0]['g'], mlp[0]['be'], mg)
    z1, s, q = _mm_first(None, gc0, mlp[0]['w'].T, mlp[0]['b'], cfa=True)
    tc1, th1 = _scale_shift(s, q, mlp[0]['g'], mlp[0]['be'], mc)

    y2, s, q = _mm(y1, mlp[1]['w'].T, mlp[1]['b'], sc1, sh1, stats=True)
    sc2, sh2 = _scale_shift(s, q, mlp[1]['g'], mlp[1]['be'], mg)
    z2, s, q = _mm(z1, mlp[1]['w'].T, mlp[1]['b'], tc1, th1, stats=True)
    tc2, th2 = _scale_shift(s, q, mlp[1]['g'], mlp[1]['be'], mc)

    y3, s, q = _mm(y2, mlp[2]['w'].T, mlp[2]['b'], sc2, sh2, stats=True)
    sc3, sh3 = _scale_shift(s, q, mlp[2]['g'], mlp[2]['be'], mg)
    z3, s, q = _mm(z2, mlp[2]['w'].T, mlp[2]['b'], tc2, th2, stats=True)
    tc3, th3 = _scale_shift(s, q, mlp[2]['g'], mlp[2]['be'], mc)

    psi_p, alp_p, gam_p, fai_p = att['psi'], att['alpha'], att['gamma'], att['fai']

    x3, p1, s, q = _mm_x3_psi1(y3, sc3, sh3, psi_p['w1'].T, psi_p['b1'])
    scp, shp = _scale_shift(s, q, psi_p['g1'], psi_p['be1'], mg)
    a1, s, q = _mm(x3, alp_p['w1'].T, alp_p['b1'], stats=True)
    sca, sha = _scale_shift(s, q, alp_p['g1'], alp_p['be1'], mg)

    psi = _mm(p1, psi_p['w2'].T, psi_p['b2'], scp, shp)
    alpha = _mm(a1, alp_p['w2'].T, alp_p['b2'], sca, sha)

    f1, s, q = _mm(z3, fai_p['w1'].T, fai_p['b1'], tc3, th3, stats=True)
    tcf, thf = _scale_shift(s, q, fai_p['g1'], fai_p['be1'], mc)
    fai = _mm(f1, fai_p['w2'].T, fai_p['b2'], tcf, thf)

    g1, s, q = _mm_gamma(fai, psi, gam_p['w1'].T, gam_p['b1'])
    scg, shg = _scale_shift(s, q, gam_p['g1'], gam_p['be1'], mg)
    y_att = _mm_attn(g1, scg, shg, alpha, gam_p['w2'].T, gam_p['b2'])

    center_xyz = gc0[:, 0:3].reshape(b, _NCTR, 3)
    center_eula = gc0[:, 3:6].reshape(b, _NCTR, 3)
    center_near = gc0[:, 6:8].reshape(b, _NCTR, 2)
    center_meta = gc0[:, 8:12].reshape(b, _NCTR, 4)
    center_fea = gc0[:, 12:76]
    new_fea_out = jnp.concatenate([center_fea, y_att],
                                  axis=1).reshape(b, _NCTR, 320)
    return center_xyz, center_eula, center_near, center_meta, new_fea_out
